# baseline (speedup 1.0000x reference)
import functools

import jax
import jax.numpy as jnp
from jax import lax
from jax.experimental import pallas as pl
from jax.experimental.pallas import tpu as pltpu
from jax.experimental.pallas import tpu_sc as plsc

N_NODES = 10000
N_EDGES = 320000
IN_C = 128
HID_C = 128
OUT_C = 64

N_PAD = 10240


def _fill_vmem_2d(ref, rows, cols, value):
    vec = jnp.full((16,), value, jnp.float32)

    def body(r, carry):
        for j in range(cols // 16):
            ref[r, pl.ds(j * 16, 16)] = vec
        return carry

    lax.fori_loop(0, rows, body, 0)


def _fill_vmem_1d(ref, n, value):
    vec = jnp.full((16,), value, jnp.float32)
    for j in range(n // 16):
        ref[pl.ds(j * 16, 16)] = vec


def _make_deg_kernel(nc, ns, iters, k):
    nw = nc * ns
    rpt = N_PAD // ns
    mesh = plsc.VectorSubcoreMesh(
        core_axis_name="c", subcore_axis_name="s", num_cores=nc)

    @functools.partial(
        pl.kernel,
        out_type=jax.ShapeDtypeStruct((nc, N_PAD), jnp.float32),
        mesh=mesh,
        scratch_types=[
            pltpu.VMEM((iters, k), jnp.int32),
            pltpu.VMEM((k,), jnp.float32),
            pltpu.VMEM((k,), jnp.float32),
            pltpu.VMEM_SHARED((N_PAD,), jnp.float32),
        ],
    )
    def deg_kernel(ei_hbm, out_hbm, didx, ones_v, zero_v, acc):
        cid = lax.axis_index("c")
        sid = lax.axis_index("s")
        wid = cid * ns + sid
        _fill_vmem_1d(ones_v, k, 1.0)
        _fill_vmem_1d(zero_v, k, 0.0)
        for q in range(rpt // k):
            pltpu.sync_copy(zero_v, acc.at[pl.ds(sid * rpt + q * k, k)])
        pltpu.sync_copy(ei_hbm.at[nw + wid], didx)
        plsc.subcore_barrier()

        def body(i, carry):
            pltpu.sync_copy(ones_v, acc.at[didx.at[i]], add=True)
            return carry

        lax.fori_loop(0, iters, body, 0)
        plsc.subcore_barrier()
        pltpu.sync_copy(acc.at[pl.ds(sid * rpt, rpt)],
                        out_hbm.at[cid, pl.ds(sid * rpt, rpt)])

    return deg_kernel


def _make_agg_kernel(c, nc, ns, iters, k):
    nw = nc * ns
    rpt = N_PAD // ns
    mesh = plsc.VectorSubcoreMesh(
        core_axis_name="c", subcore_axis_name="s", num_cores=nc)

    @functools.partial(
        pl.kernel,
        out_type=jax.ShapeDtypeStruct((nc, N_PAD, c), jnp.float32),
        mesh=mesh,
        scratch_types=[
            pltpu.VMEM((iters, k), jnp.int32),
            pltpu.VMEM((iters, k), jnp.int32),
            [pltpu.VMEM((k, c), jnp.float32)] * 4,
            [pltpu.SemaphoreType.DMA] * 4,
            [pltpu.SemaphoreType.DMA] * 4,
            pltpu.VMEM_SHARED((N_PAD, c), jnp.float32),
        ],
        compiler_params=pltpu.CompilerParams(use_tc_tiling_on_sc=False),
    )
    def agg_kernel(g_hbm, ei_hbm, out_hbm,
                   sidx, didx, rows, gsems, ssems, acc):
        cid = lax.axis_index("c")
        sid = lax.axis_index("s")
        wid = cid * ns + sid
        _fill_vmem_2d(rows[0], k, c, 0.0)
        for q in range(rpt // k):
            pltpu.sync_copy(rows[0], acc.at[pl.ds(sid * rpt + q * k, k)])
        pltpu.sync_copy(ei_hbm.at[wid], sidx)
        pltpu.sync_copy(ei_hbm.at[nw + wid], didx)
        plsc.subcore_barrier()

        def gather(j, t):
            pltpu.async_copy(g_hbm.at[sidx.at[j]], rows[t], gsems[t])

        def gather_wait(j, t):
            pltpu.make_async_copy(g_hbm.at[sidx.at[j]], rows[t],
                                  gsems[t]).wait()

        def scatter(j, t):
            pltpu.async_copy(rows[t], acc.at[didx.at[j]], ssems[t], add=True)

        def scatter_wait(j, t):
            pltpu.make_async_copy(rows[t], acc.at[didx.at[j]],
                                  ssems[t]).wait()

        gather(0, 0)
        gather(1, 1)

        def body(q, carry):
            for t in range(4):
                j = 4 * q + t
                tn = (t + 2) % 4

                @pl.when(jnp.logical_and(j >= 2, j + 2 < iters))
                def _(j=j, tn=tn):
                    scatter_wait(j - 2, tn)

                @pl.when(j + 2 < iters)
                def _(j=j, tn=tn):
                    gather(j + 2, tn)

                gather_wait(j, t)
                scatter(j, t)
            return carry

        lax.fori_loop(0, iters // 4, body, 0)
        for t in range(4):
            scatter_wait(iters - 4 + t, (iters - 4 + t) % 4)
        plsc.subcore_barrier()
        pltpu.sync_copy(acc.at[pl.ds(sid * rpt, rpt)],
                        out_hbm.at[cid, pl.ds(sid * rpt, rpt)])

    return agg_kernel


def _make_agg_kernel_wide(c, nc, ns, iters, k):
    nw = nc * ns
    rpt = N_PAD // ns
    half = iters // 2
    mesh = plsc.VectorSubcoreMesh(
        core_axis_name="c", subcore_axis_name="s", num_cores=nc)

    @functools.partial(
        pl.kernel,
        out_type=jax.ShapeDtypeStruct((nc, N_PAD, c), jnp.float32),
        mesh=mesh,
        scratch_types=[
            pltpu.VMEM((half, k), jnp.int32),
            pltpu.VMEM((half, k), jnp.int32),
            pltpu.VMEM((k, c), jnp.float32),
            pltpu.VMEM((k, c), jnp.float32),
            pltpu.SemaphoreType.DMA,
            pltpu.SemaphoreType.DMA,
            pltpu.VMEM_SHARED((N_PAD, c), jnp.float32),
        ],
        compiler_params=pltpu.CompilerParams(use_tc_tiling_on_sc=False),
    )
    def agg_kernel(g_hbm, ei_hbm, out_hbm,
                   sidx, didx, rows0, rows1, gsem0, gsem1, acc):
        cid = lax.axis_index("c")
        sid = lax.axis_index("s")
        wid = cid * ns + sid
        _fill_vmem_2d(rows0, k, c, 0.0)
        for q in range(rpt // k):
            pltpu.sync_copy(rows0, acc.at[pl.ds(sid * rpt + q * k, k)])
        plsc.subcore_barrier()

        for h in range(2):
            pltpu.sync_copy(ei_hbm.at[wid, pl.ds(h * half, half)], sidx)
            pltpu.sync_copy(ei_hbm.at[nw + wid, pl.ds(h * half, half)], didx)
            pltpu.async_copy(g_hbm.at[sidx.at[0]], rows0, gsem0)

            def body(p, carry):
                i0 = 2 * p
                i1 = i0 + 1
                pltpu.async_copy(g_hbm.at[sidx.at[i1]], rows1, gsem1)
                pltpu.make_async_copy(g_hbm.at[sidx.at[i0]], rows0,
                                      gsem0).wait()
                pltpu.sync_copy(rows0, acc.at[didx.at[i0]], add=True)

                @pl.when(i0 + 2 < half)
                def _():
                    pltpu.async_copy(g_hbm.at[sidx.at[i0 + 2]], rows0, gsem0)

                pltpu.make_async_copy(g_hbm.at[sidx.at[i1]], rows1,
                                      gsem1).wait()
                pltpu.sync_copy(rows1, acc.at[didx.at[i1]], add=True)
                return carry

            lax.fori_loop(0, half // 2, body, 0)

        plsc.subcore_barrier()
        pltpu.sync_copy(acc.at[pl.ds(sid * rpt, rpt)],
                        out_hbm.at[cid, pl.ds(sid * rpt, rpt)])

    return agg_kernel


_ROWS = 2000


def _tc1_body(x_ref, w_ref, deg_ref, g_ref):
    dinv = lax.rsqrt(deg_ref[...])
    g_ref[...] = jnp.dot(x_ref[...], w_ref[...],
                         preferred_element_type=jnp.float32) * dinv


def _tc2_body(agg_ref, g1_ref, deg_ref, b_ref, w_ref, g2_ref):
    dinv = lax.rsqrt(deg_ref[...])
    s = agg_ref[0] + agg_ref[1] + g1_ref[...]
    u = jnp.maximum(s * dinv + b_ref[...], 0.0)
    g2_ref[...] = jnp.dot(u, w_ref[...],
                          preferred_element_type=jnp.float32) * dinv


def _tc3_body(agg_ref, g2_ref, deg_ref, b_ref, z_ref):
    dinv = lax.rsqrt(deg_ref[...])
    s = agg_ref[0] + agg_ref[1] + g2_ref[...]
    z_ref[...] = jnp.maximum(s * dinv + b_ref[...], 0.0)


def _tc1(x, w1, deg2d):
    grid = (N_NODES // _ROWS,)
    return pl.pallas_call(
        _tc1_body,
        grid=grid,
        in_specs=[
            pl.BlockSpec((_ROWS, IN_C), lambda i: (i, 0)),
            pl.BlockSpec((IN_C, HID_C), lambda i: (0, 0)),
            pl.BlockSpec((_ROWS, 1), lambda i: (i, 0)),
        ],
        out_specs=pl.BlockSpec((_ROWS, HID_C), lambda i: (i, 0)),
        out_shape=jax.ShapeDtypeStruct((N_NODES, HID_C), jnp.float32),
    )(x, w1, deg2d)


def _tc2(agg1, g1, deg2d, b1_2d, w2, nc):
    grid = (N_NODES // _ROWS,)
    return pl.pallas_call(
        _tc2_body,
        grid=grid,
        in_specs=[
            pl.BlockSpec((nc, _ROWS, HID_C), lambda i: (0, i, 0)),
            pl.BlockSpec((_ROWS, HID_C), lambda i: (i, 0)),
            pl.BlockSpec((_ROWS, 1), lambda i: (i, 0)),
            pl.BlockSpec((1, HID_C), lambda i: (0, 0)),
            pl.BlockSpec((HID_C, OUT_C), lambda i: (0, 0)),
        ],
        out_specs=pl.BlockSpec((_ROWS, OUT_C), lambda i: (i, 0)),
        out_shape=jax.ShapeDtypeStruct((N_NODES, OUT_C), jnp.float32),
    )(agg1, g1, deg2d, b1_2d, w2)


def _tc3(agg2, g2, deg2d, b2_2d, nc):
    grid = (N_NODES // _ROWS,)
    return pl.pallas_call(
        _tc3_body,
        grid=grid,
        in_specs=[
            pl.BlockSpec((nc, _ROWS, OUT_C), lambda i: (0, i, 0)),
            pl.BlockSpec((_ROWS, OUT_C), lambda i: (i, 0)),
            pl.BlockSpec((_ROWS, 1), lambda i: (i, 0)),
            pl.BlockSpec((1, OUT_C), lambda i: (0, 0)),
        ],
        out_specs=pl.BlockSpec((_ROWS, OUT_C), lambda i: (i, 0)),
        out_shape=jax.ShapeDtypeStruct((N_NODES, OUT_C), jnp.float32),
    )(agg2, g2, deg2d, b2_2d)


def kernel(x, edge_index, W1, b1, W2, b2):
    info = plsc.get_sparse_core_info()
    nc, ns = info.num_cores, info.num_subcores
    nw = nc * ns
    k = 128
    iters = -(-N_EDGES // (nw * k))
    iters = (iters + 3) // 4 * 4
    e_pad = nw * iters * k

    ei = edge_index.astype(jnp.int32)
    n_fill = e_pad - N_EDGES
    fill_dst = N_NODES + jnp.arange(n_fill, dtype=jnp.int32) % (N_PAD - N_NODES)
    filler = jnp.stack([jnp.zeros((n_fill,), jnp.int32), fill_dst])
    ei3 = jnp.concatenate([ei, filler], axis=1).reshape(2 * nw, iters, k)

    degp = _make_deg_kernel(nc, ns, iters, k)(ei3)
    deg2d = (degp[:, :N_NODES].sum(axis=0) + 1.0).reshape(N_NODES, 1)

    g1 = _tc1(x, W1, deg2d)
    agg1 = _make_agg_kernel_wide(HID_C, nc, ns, iters, k)(g1, ei3)
    g2 = _tc2(agg1, g1, deg2d, b1.reshape(1, HID_C), W2, nc)
    agg2 = _make_agg_kernel(OUT_C, nc, ns, iters, k)(g2, ei3)
    z = _tc3(agg2, g2, deg2d, b2.reshape(1, OUT_C), nc)
    return z


# speedup vs baseline: 2.9141x; 2.9141x over previous
import functools

import jax
import jax.numpy as jnp
from jax import lax
from jax.experimental import pallas as pl
from jax.experimental.pallas import tpu as pltpu
from jax.experimental.pallas import tpu_sc as plsc

N_NODES = 10000
N_EDGES = 320000
IN_C = 128
HID_C = 128
OUT_C = 64

N_PAD = 10240


def _make_deg_kernel(nc, ns, iters, k):
    nw = nc * ns
    rpt = N_PAD // ns
    mesh = plsc.VectorSubcoreMesh(
        core_axis_name="c", subcore_axis_name="s", num_cores=nc)

    @functools.partial(
        pl.kernel,
        out_type=jax.ShapeDtypeStruct((nc, N_PAD), jnp.float32),
        mesh=mesh,
        scratch_types=[
            pltpu.VMEM((iters, k), jnp.int32),
            pltpu.VMEM((k,), jnp.float32),
            pltpu.VMEM_SHARED((N_PAD,), jnp.float32),
        ],
    )
    def deg_kernel(ei_hbm, ones_hbm, zeros_hbm, out_hbm, didx, ones_v, acc):
        cid = lax.axis_index("c")
        sid = lax.axis_index("s")
        wid = cid * ns + sid
        pltpu.sync_copy(zeros_hbm.at[pl.ds(sid * rpt, rpt)],
                        acc.at[pl.ds(sid * rpt, rpt)])
        pltpu.sync_copy(ei_hbm.at[nw + wid], didx)
        pltpu.sync_copy(ones_hbm, ones_v)
        plsc.subcore_barrier()

        def body(i, carry):
            pltpu.sync_copy(ones_v, acc.at[didx.at[i]], add=True)
            return carry

        lax.fori_loop(0, iters, body, 0)
        plsc.subcore_barrier()
        pltpu.sync_copy(acc.at[pl.ds(sid * rpt, rpt)],
                        out_hbm.at[cid, pl.ds(sid * rpt, rpt)])

    return deg_kernel


def _make_agg_kernel(c, nc, ns, iters, k):
    nw = nc * ns
    rpt = N_PAD // ns
    mesh = plsc.VectorSubcoreMesh(
        core_axis_name="c", subcore_axis_name="s", num_cores=nc)

    @functools.partial(
        pl.kernel,
        out_type=jax.ShapeDtypeStruct((nc, N_PAD, c), jnp.float32),
        mesh=mesh,
        scratch_types=[
            pltpu.VMEM((iters, k), jnp.int32),
            pltpu.VMEM((iters, k), jnp.int32),
            [pltpu.VMEM((k, c), jnp.float32)] * 4,
            [pltpu.SemaphoreType.DMA] * 4,
            [pltpu.SemaphoreType.DMA] * 4,
            pltpu.VMEM_SHARED((N_PAD, c), jnp.float32),
        ],
        compiler_params=pltpu.CompilerParams(use_tc_tiling_on_sc=False),
    )
    def agg_kernel(g_hbm, ei_hbm, zeros_hbm, out_hbm,
                   sidx, didx, rows, gsems, ssems, acc):
        cid = lax.axis_index("c")
        sid = lax.axis_index("s")
        wid = cid * ns + sid
        pltpu.sync_copy(zeros_hbm.at[pl.ds(sid * rpt, rpt)],
                        acc.at[pl.ds(sid * rpt, rpt)])
        pltpu.sync_copy(ei_hbm.at[wid], sidx)
        pltpu.sync_copy(ei_hbm.at[nw + wid], didx)
        plsc.subcore_barrier()

        def gather(j, t):
            pltpu.async_copy(g_hbm.at[sidx.at[j]], rows[t], gsems[t])

        def gather_wait(j, t):
            pltpu.make_async_copy(g_hbm.at[sidx.at[j]], rows[t],
                                  gsems[t]).wait()

        def scatter(j, t):
            pltpu.async_copy(rows[t], acc.at[didx.at[j]], ssems[t], add=True)

        def scatter_wait(j, t):
            pltpu.make_async_copy(rows[t], acc.at[didx.at[j]],
                                  ssems[t]).wait()

        gather(0, 0)
        gather(1, 1)

        def body(q, carry):
            for t in range(4):
                j = 4 * q + t
                tn = (t + 2) % 4

                @pl.when(jnp.logical_and(j >= 2, j + 2 < iters))
                def _(j=j, tn=tn):
                    scatter_wait(j - 2, tn)

                @pl.when(j + 2 < iters)
                def _(j=j, tn=tn):
                    gather(j + 2, tn)

                gather_wait(j, t)
                scatter(j, t)
            return carry

        lax.fori_loop(0, iters // 4, body, 0)
        for t in range(4):
            scatter_wait(iters - 4 + t, (iters - 4 + t) % 4)
        plsc.subcore_barrier()
        pltpu.sync_copy(acc.at[pl.ds(sid * rpt, rpt)],
                        out_hbm.at[cid, pl.ds(sid * rpt, rpt)])

    return agg_kernel


_ROWS = 2000


def _tc1_body(x_ref, w_ref, deg_ref, g_ref):
    dinv = lax.rsqrt(deg_ref[...])
    g_ref[...] = jnp.dot(x_ref[...], w_ref[...],
                         preferred_element_type=jnp.float32) * dinv


def _tc2_body(agg_ref, g1_ref, deg_ref, b_ref, w_ref, g2_ref):
    dinv = lax.rsqrt(deg_ref[...])
    s = agg_ref[0] + agg_ref[1] + g1_ref[...]
    u = jnp.maximum(s * dinv + b_ref[...], 0.0)
    g2_ref[...] = jnp.dot(u, w_ref[...],
                          preferred_element_type=jnp.float32) * dinv


def _tc3_body(agg_ref, g2_ref, deg_ref, b_ref, z_ref):
    dinv = lax.rsqrt(deg_ref[...])
    s = agg_ref[0] + agg_ref[1] + g2_ref[...]
    z_ref[...] = jnp.maximum(s * dinv + b_ref[...], 0.0)


def _tc1(x, w1, deg2d):
    grid = (N_NODES // _ROWS,)
    return pl.pallas_call(
        _tc1_body,
        grid=grid,
        in_specs=[
            pl.BlockSpec((_ROWS, IN_C), lambda i: (i, 0)),
            pl.BlockSpec((IN_C, HID_C), lambda i: (0, 0)),
            pl.BlockSpec((_ROWS, 1), lambda i: (i, 0)),
        ],
        out_specs=pl.BlockSpec((_ROWS, HID_C), lambda i: (i, 0)),
        out_shape=jax.ShapeDtypeStruct((N_NODES, HID_C), jnp.float32),
    )(x, w1, deg2d)


def _tc2(agg1, g1, deg2d, b1_2d, w2, nc):
    grid = (N_NODES // _ROWS,)
    return pl.pallas_call(
        _tc2_body,
        grid=grid,
        in_specs=[
            pl.BlockSpec((nc, _ROWS, HID_C), lambda i: (0, i, 0)),
            pl.BlockSpec((_ROWS, HID_C), lambda i: (i, 0)),
            pl.BlockSpec((_ROWS, 1), lambda i: (i, 0)),
            pl.BlockSpec((1, HID_C), lambda i: (0, 0)),
            pl.BlockSpec((HID_C, OUT_C), lambda i: (0, 0)),
        ],
        out_specs=pl.BlockSpec((_ROWS, OUT_C), lambda i: (i, 0)),
        out_shape=jax.ShapeDtypeStruct((N_NODES, OUT_C), jnp.float32),
    )(agg1, g1, deg2d, b1_2d, w2)


def _tc3(agg2, g2, deg2d, b2_2d, nc):
    grid = (N_NODES // _ROWS,)
    return pl.pallas_call(
        _tc3_body,
        grid=grid,
        in_specs=[
            pl.BlockSpec((nc, _ROWS, OUT_C), lambda i: (0, i, 0)),
            pl.BlockSpec((_ROWS, OUT_C), lambda i: (i, 0)),
            pl.BlockSpec((_ROWS, 1), lambda i: (i, 0)),
            pl.BlockSpec((1, OUT_C), lambda i: (0, 0)),
        ],
        out_specs=pl.BlockSpec((_ROWS, OUT_C), lambda i: (i, 0)),
        out_shape=jax.ShapeDtypeStruct((N_NODES, OUT_C), jnp.float32),
    )(agg2, g2, deg2d, b2_2d)


def kernel(x, edge_index, W1, b1, W2, b2):
    info = plsc.get_sparse_core_info()
    nc, ns = info.num_cores, info.num_subcores
    nw = nc * ns
    epw = N_EDGES // nw
    k_h = 50
    k_o = 125
    it_h = epw // k_h
    it_o = epw // k_o

    ei = edge_index.astype(jnp.int32)
    ei_h = ei.reshape(2 * nw, it_h, k_h)
    ei_o = ei.reshape(2 * nw, it_o, k_o)

    ones_k = jnp.ones((k_o,), jnp.float32)
    zeros_1 = jnp.zeros((N_PAD,), jnp.float32)
    zeros_h = jnp.zeros((N_PAD, HID_C), jnp.float32)
    zeros_o = jnp.zeros((N_PAD, OUT_C), jnp.float32)

    degp = _make_deg_kernel(nc, ns, it_o, k_o)(ei_o, ones_k, zeros_1)
    deg2d = (degp[:, :N_NODES].sum(axis=0) + 1.0).reshape(N_NODES, 1)

    g1 = _tc1(x, W1, deg2d)
    agg1 = _make_agg_kernel(HID_C, nc, ns, it_h, k_h)(g1, ei_h, zeros_h)
    g2 = _tc2(agg1, g1, deg2d, b1.reshape(1, HID_C), W2, nc)
    agg2 = _make_agg_kernel(OUT_C, nc, ns, it_o, k_o)(g2, ei_o, zeros_o)
    z = _tc3(agg2, g2, deg2d, b2.reshape(1, OUT_C), nc)
    return z
